# baseline (speedup 1.0000x reference)
import functools

import numpy as np

import jax
import jax.numpy as jnp
from jax.experimental import pallas as pl
from jax.experimental.pallas import tpu as pltpu

N = 16384
K = 1024
D = 64
R = 4096
BETA = 0.25
DECAY = 0.99
USAGE_LAMBDA = 0.005


def _make_reinit_table():
    r = jax.random.normal(jax.random.key(1), (K, D), dtype=jnp.float32)
    rn = jnp.linalg.norm(r, axis=-1, keepdims=True)
    return r / jnp.clip(rn, 1e-12, None) * 0.1


try:
    with jax.default_device(jax.local_devices(backend="cpu")[0]):
        _R_TABLE_T = np.asarray(_make_reinit_table()).T.copy()
except Exception:
    _R_TABLE_T = None


def _vq_body(xt_ref, cbt_ref, emac_ref, emast_ref, rt_ref,
             quantt_ref, idx_ref, loss_ref, ent_ref, perp_ref, counts_ref,
             newcbt_ref, newcount_ref, newsumt_ref,
             sumvec_acc, cbn_scratch, sse_acc):
    i = pl.program_id(0)

    @pl.when(i == 0)
    def _init():
        sumvec_acc[...] = jnp.zeros_like(sumvec_acc)
        sse_acc[0] = jnp.float32(0.0)
        cbt = cbt_ref[...]
        cbnt = cbt / jnp.clip(
            jnp.sqrt(jnp.sum(cbt * cbt, axis=0, keepdims=True)), 1e-12, None)
        cbn_scratch[...] = jnp.transpose(cbnt)

    xt = xt_ref[...]
    s2 = jnp.sum(xt * xt, axis=0, keepdims=True)
    xnt = xt / jnp.clip(jnp.sqrt(s2), 1e-12, None)
    sse_acc[0] += jnp.sum(s2)
    sims = jax.lax.dot_general(cbn_scratch[...], xnt,
                               (((1,), (0,)), ((), ())),
                               preferred_element_type=jnp.float32)
    dist = 1.0 - sims
    m = jnp.min(dist, axis=0, keepdims=True)
    iota0 = jax.lax.broadcasted_iota(jnp.int32, (K, R), 0)
    idxmat = jnp.where(dist == m, iota0, jnp.int32(K))
    idx = jnp.min(idxmat, axis=0)
    one_hot = (iota0 == idx[None, :]).astype(jnp.float32)
    quantt = jax.lax.dot_general(cbt_ref[...], one_hot,
                                 (((1,), (0,)), ((), ())),
                                 preferred_element_type=jnp.float32)
    quantt_ref[...] = quantt
    idx_ref[...] = idx

    xat = jnp.concatenate([xt, jnp.ones((8, R), dtype=jnp.float32)],
                          axis=0)
    sumvec_acc[...] += jax.lax.dot_general(
        xat, one_hot, (((1,), (1,)), ((), ())),
        preferred_element_type=jnp.float32)

    @pl.when(i == pl.num_programs(0) - 1)
    def _finalize():
        counts_row = sumvec_acc[D:D + 1, :]
        total = jnp.sum(counts_row)
        p = counts_row / (total + 1e-6)
        ent = -jnp.sum(p * jnp.log(p + 1e-12))
        logC = jnp.log(jnp.float32(K) + 1e-12)
        cbt = cbt_ref[...]
        cross = jnp.sum(cbt * sumvec_acc[:D, :])
        quad = jnp.sum(jnp.sum(cbt * cbt, axis=0, keepdims=True) * counts_row)
        sse = sse_acc[0] - 2.0 * cross + quad
        commit = BETA * sse / jnp.float32(N * D)
        loss_ref[...] = jnp.full((1, 1), commit + USAGE_LAMBDA * (logC - ent),
                                 dtype=jnp.float32)
        ent_ref[...] = jnp.full((1, 1), ent, dtype=jnp.float32)
        perp_ref[...] = jnp.full((1, 1), jnp.exp(ent), dtype=jnp.float32)
        counts_ref[...] = counts_row.reshape(K)
        emac_row = emac_ref[...].reshape(1, K)
        new_count = emac_row * DECAY + counts_row * (1.0 - DECAY)
        new_sumt = (emast_ref[...] * DECAY
                    + sumvec_acc[:D, :] * (1.0 - DECAY))
        n = new_count + 1e-5
        new_cbt = new_sumt / n
        dead = new_count < 0.001
        rt = rt_ref[...]
        newcbt_ref[...] = jnp.where(dead, rt, new_cbt)
        newsumt_ref[...] = jnp.where(dead, rt, new_sumt)
        newcount_ref[...] = jnp.where(dead, jnp.float32(1.0),
                                      new_count).reshape(K)


@functools.partial(jax.jit)
def kernel(x, codebook, ema_count, ema_sum):
    grid = N // R
    rt = (jnp.transpose(_make_reinit_table()) if _R_TABLE_T is None
          else jnp.asarray(_R_TABLE_T))

    out_shapes = (
        jax.ShapeDtypeStruct((D, N), jnp.float32),
        jax.ShapeDtypeStruct((N,), jnp.int32),
        jax.ShapeDtypeStruct((1, 1), jnp.float32),
        jax.ShapeDtypeStruct((1, 1), jnp.float32),
        jax.ShapeDtypeStruct((1, 1), jnp.float32),
        jax.ShapeDtypeStruct((K,), jnp.float32),
        jax.ShapeDtypeStruct((D, K), jnp.float32),
        jax.ShapeDtypeStruct((K,), jnp.float32),
        jax.ShapeDtypeStruct((D, K), jnp.float32),
    )
    in_specs = [
        pl.BlockSpec((D, R), lambda i: (0, i)),
        pl.BlockSpec((D, K), lambda i: (0, 0)),
        pl.BlockSpec((K,), lambda i: (0,)),
        pl.BlockSpec((D, K), lambda i: (0, 0)),
        pl.BlockSpec((D, K), lambda i: (0, 0)),
    ]
    out_specs = (
        pl.BlockSpec((D, R), lambda i: (0, i)),
        pl.BlockSpec((R,), lambda i: (i,)),
        pl.BlockSpec((1, 1), lambda i: (0, 0)),
        pl.BlockSpec((1, 1), lambda i: (0, 0)),
        pl.BlockSpec((1, 1), lambda i: (0, 0)),
        pl.BlockSpec((K,), lambda i: (0,)),
        pl.BlockSpec((D, K), lambda i: (0, 0)),
        pl.BlockSpec((K,), lambda i: (0,)),
        pl.BlockSpec((D, K), lambda i: (0, 0)),
    )
    outs = pl.pallas_call(
        _vq_body,
        grid=(grid,),
        in_specs=in_specs,
        out_specs=out_specs,
        out_shape=out_shapes,
        scratch_shapes=[
            pltpu.VMEM((D + 8, K), jnp.float32),
            pltpu.VMEM((K, D), jnp.float32),
            pltpu.SMEM((1,), jnp.float32),
        ],
    )(x.T, codebook.T, ema_count, ema_sum.T, rt)
    (quantt, idx, vq, ent, perp, counts, new_cbt, new_count, new_sumt) = outs
    return (quantt.T, idx, vq[0, 0], ent[0, 0], perp[0, 0],
            counts, new_cbt.T, new_count, new_sumt.T)


# speedup vs baseline: 1.0100x; 1.0100x over previous
import functools

import numpy as np

import jax
import jax.numpy as jnp
from jax.experimental import pallas as pl
from jax.experimental.pallas import tpu as pltpu

N = 16384
K = 1024
D = 64
R = 2048
BETA = 0.25
DECAY = 0.99
USAGE_LAMBDA = 0.005


def _make_reinit_table():
    r = jax.random.normal(jax.random.key(1), (K, D), dtype=jnp.float32)
    rn = jnp.linalg.norm(r, axis=-1, keepdims=True)
    return r / jnp.clip(rn, 1e-12, None) * 0.1


try:
    with jax.default_device(jax.local_devices(backend="cpu")[0]):
        _R_TABLE_T = np.asarray(_make_reinit_table()).T.copy()
except Exception:
    _R_TABLE_T = None


def _vq_body(xt_ref, cbt_ref, emac_ref, emast_ref, rt_ref,
             quantt_ref, idx_ref, loss_ref, ent_ref, perp_ref, counts_ref,
             newcbt_ref, newcount_ref, newsumt_ref,
             sumvec_acc, cbn_scratch, sse_acc):
    i = pl.program_id(0)

    @pl.when(i == 0)
    def _init():
        sumvec_acc[...] = jnp.zeros_like(sumvec_acc)
        sse_acc[0] = jnp.float32(0.0)
        cbt = cbt_ref[...]
        cbnt = cbt / jnp.clip(
            jnp.sqrt(jnp.sum(cbt * cbt, axis=0, keepdims=True)), 1e-12, None)
        cbn_scratch[...] = jnp.transpose(cbnt)

    xt = xt_ref[...]
    s2 = jnp.sum(xt * xt, axis=0, keepdims=True)
    xnt = xt / jnp.clip(jnp.sqrt(s2), 1e-12, None)
    sse_acc[0] += jnp.sum(s2)
    sims = jax.lax.dot_general(cbn_scratch[...], xnt,
                               (((1,), (0,)), ((), ())),
                               preferred_element_type=jnp.float32)
    dist = 1.0 - sims
    m = jnp.min(dist, axis=0, keepdims=True)
    iota0 = jax.lax.broadcasted_iota(jnp.int32, (K, R), 0)
    idxmat = jnp.where(dist == m, iota0, jnp.int32(K))
    idx = jnp.min(idxmat, axis=0)
    one_hot = (iota0 == idx[None, :]).astype(jnp.float32)
    quantt = jax.lax.dot_general(cbt_ref[...], one_hot,
                                 (((1,), (0,)), ((), ())),
                                 preferred_element_type=jnp.float32)
    quantt_ref[...] = quantt
    idx_ref[...] = idx

    xat = jnp.concatenate([xt, jnp.ones((8, R), dtype=jnp.float32)],
                          axis=0)
    sumvec_acc[...] += jax.lax.dot_general(
        xat, one_hot, (((1,), (1,)), ((), ())),
        preferred_element_type=jnp.float32)

    @pl.when(i == pl.num_programs(0) - 1)
    def _finalize():
        counts_row = sumvec_acc[D:D + 1, :]
        total = jnp.sum(counts_row)
        p = counts_row / (total + 1e-6)
        ent = -jnp.sum(p * jnp.log(p + 1e-12))
        logC = jnp.log(jnp.float32(K) + 1e-12)
        cbt = cbt_ref[...]
        cross = jnp.sum(cbt * sumvec_acc[:D, :])
        quad = jnp.sum(jnp.sum(cbt * cbt, axis=0, keepdims=True) * counts_row)
        sse = sse_acc[0] - 2.0 * cross + quad
        commit = BETA * sse / jnp.float32(N * D)
        loss_ref[...] = jnp.full((1, 1), commit + USAGE_LAMBDA * (logC - ent),
                                 dtype=jnp.float32)
        ent_ref[...] = jnp.full((1, 1), ent, dtype=jnp.float32)
        perp_ref[...] = jnp.full((1, 1), jnp.exp(ent), dtype=jnp.float32)
        counts_ref[...] = counts_row.reshape(K)
        emac_row = emac_ref[...].reshape(1, K)
        new_count = emac_row * DECAY + counts_row * (1.0 - DECAY)
        new_sumt = (emast_ref[...] * DECAY
                    + sumvec_acc[:D, :] * (1.0 - DECAY))
        n = new_count + 1e-5
        new_cbt = new_sumt / n
        dead = new_count < 0.001
        rt = rt_ref[...]
        newcbt_ref[...] = jnp.where(dead, rt, new_cbt)
        newsumt_ref[...] = jnp.where(dead, rt, new_sumt)
        newcount_ref[...] = jnp.where(dead, jnp.float32(1.0),
                                      new_count).reshape(K)


@functools.partial(jax.jit)
def kernel(x, codebook, ema_count, ema_sum):
    grid = N // R
    rt = (jnp.transpose(_make_reinit_table()) if _R_TABLE_T is None
          else jnp.asarray(_R_TABLE_T))

    out_shapes = (
        jax.ShapeDtypeStruct((D, N), jnp.float32),
        jax.ShapeDtypeStruct((N,), jnp.int32),
        jax.ShapeDtypeStruct((1, 1), jnp.float32),
        jax.ShapeDtypeStruct((1, 1), jnp.float32),
        jax.ShapeDtypeStruct((1, 1), jnp.float32),
        jax.ShapeDtypeStruct((K,), jnp.float32),
        jax.ShapeDtypeStruct((D, K), jnp.float32),
        jax.ShapeDtypeStruct((K,), jnp.float32),
        jax.ShapeDtypeStruct((D, K), jnp.float32),
    )
    in_specs = [
        pl.BlockSpec((D, R), lambda i: (0, i)),
        pl.BlockSpec((D, K), lambda i: (0, 0)),
        pl.BlockSpec((K,), lambda i: (0,)),
        pl.BlockSpec((D, K), lambda i: (0, 0)),
        pl.BlockSpec((D, K), lambda i: (0, 0)),
    ]
    out_specs = (
        pl.BlockSpec((D, R), lambda i: (0, i)),
        pl.BlockSpec((R,), lambda i: (i,)),
        pl.BlockSpec((1, 1), lambda i: (0, 0)),
        pl.BlockSpec((1, 1), lambda i: (0, 0)),
        pl.BlockSpec((1, 1), lambda i: (0, 0)),
        pl.BlockSpec((K,), lambda i: (0,)),
        pl.BlockSpec((D, K), lambda i: (0, 0)),
        pl.BlockSpec((K,), lambda i: (0,)),
        pl.BlockSpec((D, K), lambda i: (0, 0)),
    )
    outs = pl.pallas_call(
        _vq_body,
        grid=(grid,),
        in_specs=in_specs,
        out_specs=out_specs,
        out_shape=out_shapes,
        scratch_shapes=[
            pltpu.VMEM((D + 8, K), jnp.float32),
            pltpu.VMEM((K, D), jnp.float32),
            pltpu.SMEM((1,), jnp.float32),
        ],
    )(x.T, codebook.T, ema_count, ema_sum.T, rt)
    (quantt, idx, vq, ent, perp, counts, new_cbt, new_count, new_sumt) = outs
    return (quantt.T, idx, vq[0, 0], ent[0, 0], perp[0, 0],
            counts, new_cbt.T, new_count, new_sumt.T)
